# baseline (speedup 1.0000x reference)
import jax
import jax.numpy as jnp
from jax import lax
from jax.experimental import pallas as pl
from jax.experimental.pallas import tpu as pltpu
from jax.experimental.pallas import tpu_sc as plsc

_DIMS = [119, 5, 12, 12, 10, 6, 6, 2, 2]
_K = 8
_D = 248
_DOUT = 256
_N = 100000
_XW = 32

_NC = 2
_NS = 16
_NW = _NC * _NS
_RPW = _N // _NW
_CHUNK = 96
_NCHUNK = 33

_PAIRS = [(2, 8), (3, 7), (4, 1), (5, 6)]
_SECROWS = [_DIMS[0]] + [_DIMS[a] * _DIMS[b] for a, b in _PAIRS]
_SSTART = [0]
for _r in _SECROWS[:-1]:
    _SSTART.append(_SSTART[-1] + _r)
_TROWS = sum(_SECROWS)


def _compute_chunk(x_v, t_v, out_v, iota):
    iota32 = iota * _XW
    iota256 = iota * _DOUT
    for g in range(6):
        rb = g * 16
        xb = rb * _XW
        ob = iota256 + rb * _DOUT

        def gat(col):
            return plsc.load_gather(x_v, [iota32 + (xb + col)])

        for c in range(_K):
            plsc.store_scatter(out_v, [ob + c], gat(c))

        ci = [gat(_K + i).astype(jnp.int32) for i in range(9)]
        addrs = [ci[0] * _D]
        for p, (a, b) in enumerate(_PAIRS):
            addrs.append(
                ci[a] * (_DIMS[b] * _D) + ci[b] * _D + _SSTART[1 + p] * _D
            )
        ob8 = ob + _K

        def col_body(blk, carry):
            for u in range(8):
                cc = blk * 8 + u
                v = None
                for k in range(5):
                    gk = plsc.load_gather(t_v, [addrs[k] + cc])
                    v = gk if v is None else v + gk
                plsc.store_scatter(out_v, [ob8 + cc], v)
            return carry

        lax.fori_loop(0, _D // 8, col_body, 0)


def _sc_body(x_hbm, t_hbm, out_hbm, xv0, xv1, ov0, ov1, t_v, sx0, sx1, so0, so1):
    wid = lax.axis_index("s") * _NC + lax.axis_index("c")
    pltpu.sync_copy(t_hbm, t_v)
    iota = lax.iota(jnp.int32, 16)
    w0 = wid * _RPW
    xbufs = [xv0, xv1]
    obufs = [ov0, ov1]
    sxs = [sx0, sx1]
    sos = [so0, so1]

    def base_of(t):
        return jnp.minimum(w0 + t * _CHUNK, _N - _CHUNK)

    nx = _CHUNK * _XW
    no = _CHUNK * _DOUT
    pltpu.async_copy(x_hbm.at[pl.ds(base_of(0) * _XW, nx)], xv0, sx0)

    @pl.loop(0, _NCHUNK, step=2)
    def _(tt):
        for b in range(2):
            t = tt + b

            @pl.when(t < _NCHUNK)
            def _():
                base = base_of(t)
                pltpu.make_async_copy(
                    x_hbm.at[pl.ds(base * _XW, nx)], xbufs[b], sxs[b]
                ).wait()

                @pl.when(t + 1 < _NCHUNK)
                def _():
                    nb = base_of(t + 1)
                    pltpu.async_copy(
                        x_hbm.at[pl.ds(nb * _XW, nx)], xbufs[1 - b], sxs[1 - b]
                    )

                @pl.when(t >= 2)
                def _():
                    pltpu.make_async_copy(
                        obufs[b], out_hbm.at[pl.ds(0, no)], sos[b]
                    ).wait()

                _compute_chunk(xbufs[b], t_v, obufs[b], iota)
                pltpu.async_copy(obufs[b], out_hbm.at[pl.ds(base * _DOUT, no)], sos[b])

    pltpu.make_async_copy(ov1, out_hbm.at[pl.ds(0, no)], so1).wait()
    pltpu.make_async_copy(ov0, out_hbm.at[pl.ds(0, no)], so0).wait()


@jax.jit
def _run(xp_flat, t_flat):
    mesh = plsc.VectorSubcoreMesh(core_axis_name="c", subcore_axis_name="s")
    f = pl.kernel(
        _sc_body,
        mesh=mesh,
        compiler_params=pltpu.CompilerParams(needs_layout_passes=False),
        out_type=jax.ShapeDtypeStruct((_N * _DOUT,), jnp.float32),
        scratch_types=[
            pltpu.VMEM((_CHUNK * _XW,), jnp.float32),
            pltpu.VMEM((_CHUNK * _XW,), jnp.float32),
            pltpu.VMEM((_CHUNK * _DOUT,), jnp.float32),
            pltpu.VMEM((_CHUNK * _DOUT,), jnp.float32),
            pltpu.VMEM((_TROWS * _D,), jnp.float32),
            pltpu.SemaphoreType.DMA,
            pltpu.SemaphoreType.DMA,
            pltpu.SemaphoreType.DMA,
            pltpu.SemaphoreType.DMA,
        ],
    )
    return f(xp_flat, t_flat)


def kernel(x, W0, W1, W2, W3, W4, W5, W6, W7, W8):
    Ws = [W0, W1, W2, W3, W4, W5, W6, W7, W8]
    secs = [W0]
    for a, b in _PAIRS:
        secs.append((Ws[a][:, None, :] + Ws[b][None, :, :]).reshape(-1, _D))
    table = jnp.concatenate(secs, axis=0)
    xp = jnp.pad(x, ((0, 0), (0, _XW - x.shape[1])))
    out = _run(xp.reshape(-1), table.reshape(-1))
    return out.reshape(_N, _DOUT)


# speedup vs baseline: 1.7018x; 1.7018x over previous
import jax
import jax.numpy as jnp
from jax import lax
from jax.experimental import pallas as pl
from jax.experimental.pallas import tpu as pltpu
from jax.experimental.pallas import tpu_sc as plsc

_DIMS = [119, 5, 12, 12, 10, 6, 6, 2, 2]
_K = 8
_D = 248
_DOUT = 256
_N = 100000
_XW = 32

_NC = 2
_NS = 16
_NW = _NC * _NS
_RPW = _N // _NW
_CHUNK = 96
_NCHUNK = 33

_PAIRS = [(2, 8), (3, 7), (4, 1), (5, 6)]
_SECROWS = [_DIMS[0]] + [_DIMS[a] * _DIMS[b] for a, b in _PAIRS]
_SSTART = [0]
for _r in _SECROWS[:-1]:
    _SSTART.append(_SSTART[-1] + _r)
_TROWS = sum(_SECROWS)


def _compute_chunk(x_v, t_v, out_v, iota):
    iota32 = iota * _XW
    iota256 = iota * _DOUT
    for g in range(6):
        rb = g * 16
        xb = rb * _XW
        ob = iota256 + rb * _DOUT

        def gat(col):
            return plsc.load_gather(x_v, [iota32 + (xb + col)])

        for c in range(_K):
            plsc.store_scatter(out_v, [ob + c], gat(c))

        ci = [gat(_K + i).astype(jnp.int32) for i in range(9)]
        addrs = [ci[0] * _D]
        for p, (a, b) in enumerate(_PAIRS):
            addrs.append(
                ci[a] * (_DIMS[b] * _D) + ci[b] * _D + _SSTART[1 + p] * _D
            )
        ob8 = ob + _K

        @plsc.parallel_loop(0, _D, unroll=8)
        def _(cc):
            v = None
            for k in range(5):
                gk = plsc.load_gather(t_v, [addrs[k] + cc])
                v = gk if v is None else v + gk
            plsc.store_scatter(out_v, [ob8 + cc], v)


def _sc_body(x_hbm, t_hbm, out_hbm, xv0, xv1, ov0, ov1, t_v, sx0, sx1, so0, so1):
    wid = lax.axis_index("s") * _NC + lax.axis_index("c")
    pltpu.sync_copy(t_hbm, t_v)
    iota = lax.iota(jnp.int32, 16)
    w0 = wid * _RPW
    xbufs = [xv0, xv1]
    obufs = [ov0, ov1]
    sxs = [sx0, sx1]
    sos = [so0, so1]

    def base_of(t):
        return jnp.minimum(w0 + t * _CHUNK, _N - _CHUNK)

    nx = _CHUNK * _XW
    no = _CHUNK * _DOUT
    pltpu.async_copy(x_hbm.at[pl.ds(base_of(0) * _XW, nx)], xv0, sx0)

    @pl.loop(0, _NCHUNK, step=2)
    def _(tt):
        for b in range(2):
            t = tt + b

            @pl.when(t < _NCHUNK)
            def _():
                base = base_of(t)
                pltpu.make_async_copy(
                    x_hbm.at[pl.ds(base * _XW, nx)], xbufs[b], sxs[b]
                ).wait()

                @pl.when(t + 1 < _NCHUNK)
                def _():
                    nb = base_of(t + 1)
                    pltpu.async_copy(
                        x_hbm.at[pl.ds(nb * _XW, nx)], xbufs[1 - b], sxs[1 - b]
                    )

                @pl.when(t >= 2)
                def _():
                    pltpu.make_async_copy(
                        obufs[b], out_hbm.at[pl.ds(0, no)], sos[b]
                    ).wait()

                _compute_chunk(xbufs[b], t_v, obufs[b], iota)
                pltpu.async_copy(obufs[b], out_hbm.at[pl.ds(base * _DOUT, no)], sos[b])

    pltpu.make_async_copy(ov1, out_hbm.at[pl.ds(0, no)], so1).wait()
    pltpu.make_async_copy(ov0, out_hbm.at[pl.ds(0, no)], so0).wait()


@jax.jit
def _run(xp_flat, t_flat):
    mesh = plsc.VectorSubcoreMesh(core_axis_name="c", subcore_axis_name="s")
    f = pl.kernel(
        _sc_body,
        mesh=mesh,
        compiler_params=pltpu.CompilerParams(needs_layout_passes=False),
        out_type=jax.ShapeDtypeStruct((_N * _DOUT,), jnp.float32),
        scratch_types=[
            pltpu.VMEM((_CHUNK * _XW,), jnp.float32),
            pltpu.VMEM((_CHUNK * _XW,), jnp.float32),
            pltpu.VMEM((_CHUNK * _DOUT,), jnp.float32),
            pltpu.VMEM((_CHUNK * _DOUT,), jnp.float32),
            pltpu.VMEM((_TROWS * _D,), jnp.float32),
            pltpu.SemaphoreType.DMA,
            pltpu.SemaphoreType.DMA,
            pltpu.SemaphoreType.DMA,
            pltpu.SemaphoreType.DMA,
        ],
    )
    return f(xp_flat, t_flat)


def kernel(x, W0, W1, W2, W3, W4, W5, W6, W7, W8):
    Ws = [W0, W1, W2, W3, W4, W5, W6, W7, W8]
    secs = [W0]
    for a, b in _PAIRS:
        secs.append((Ws[a][:, None, :] + Ws[b][None, :, :]).reshape(-1, _D))
    table = jnp.concatenate(secs, axis=0)
    xp = jnp.pad(x, ((0, 0), (0, _XW - x.shape[1])))
    out = _run(xp.reshape(-1), table.reshape(-1))
    return out.reshape(_N, _DOUT)


# speedup vs baseline: 2.0444x; 1.2013x over previous
import jax
import jax.numpy as jnp
from jax import lax
from jax.experimental import pallas as pl
from jax.experimental.pallas import tpu as pltpu
from jax.experimental.pallas import tpu_sc as plsc

_DIMS = [119, 5, 12, 12, 10, 6, 6, 2, 2]
_K = 8
_D = 248
_DOUT = 256
_N = 100000
_XW = 32

_NC = 2
_NS = 16
_NW = _NC * _NS
_RPW = _N // _NW
_CHUNK = 96
_NCHUNK = 33

_PAIRS = [(2, 8), (3, 7), (4, 1), (5, 6)]
_SECROWS = [_DIMS[0]] + [_DIMS[a] * _DIMS[b] for a, b in _PAIRS]
_SSTART = [0]
for _r in _SECROWS[:-1]:
    _SSTART.append(_SSTART[-1] + _r)
_TROWS = sum(_SECROWS)
_TS = _D + 1


def _compute_chunk(x_v, t_v, out_v, iota):
    iota32 = iota * _XW
    iota256 = iota * _DOUT
    for g in range(6):
        rb = g * 16
        xb = rb * _XW
        ob = iota256 + rb * _DOUT

        def gat(col):
            return plsc.load_gather(x_v, [iota32 + (xb + col)])

        for c in range(_K):
            plsc.store_scatter(out_v, [ob + c], gat(c))

        ci = [gat(_K + i).astype(jnp.int32) for i in range(9)]
        addrs = [ci[0] * _TS]
        for p, (a, b) in enumerate(_PAIRS):
            addrs.append(
                ci[a] * (_DIMS[b] * _TS) + ci[b] * _TS + _SSTART[1 + p] * _TS
            )
        ob8 = ob + _K

        @plsc.parallel_loop(0, _D, unroll=8)
        def _(cc):
            v = None
            for k in range(5):
                gk = plsc.load_gather(t_v, [addrs[k] + cc])
                v = gk if v is None else v + gk
            plsc.store_scatter(out_v, [ob8 + cc], v)


def _sc_body(x_hbm, t_hbm, out_hbm, xv0, xv1, ov0, ov1, t_v, sx0, sx1, so0, so1):
    wid = lax.axis_index("s") * _NC + lax.axis_index("c")
    pltpu.sync_copy(t_hbm, t_v)
    iota = lax.iota(jnp.int32, 16)
    w0 = wid * _RPW
    xbufs = [xv0, xv1]
    obufs = [ov0, ov1]
    sxs = [sx0, sx1]
    sos = [so0, so1]

    def base_of(t):
        return jnp.minimum(w0 + t * _CHUNK, _N - _CHUNK)

    nx = _CHUNK * _XW
    no = _CHUNK * _DOUT
    pltpu.async_copy(x_hbm.at[pl.ds(base_of(0) * _XW, nx)], xv0, sx0)

    @pl.loop(0, _NCHUNK, step=2)
    def _(tt):
        for b in range(2):
            t = tt + b

            @pl.when(t < _NCHUNK)
            def _():
                base = base_of(t)
                pltpu.make_async_copy(
                    x_hbm.at[pl.ds(base * _XW, nx)], xbufs[b], sxs[b]
                ).wait()

                @pl.when(t + 1 < _NCHUNK)
                def _():
                    nb = base_of(t + 1)
                    pltpu.async_copy(
                        x_hbm.at[pl.ds(nb * _XW, nx)], xbufs[1 - b], sxs[1 - b]
                    )

                @pl.when(t >= 2)
                def _():
                    pltpu.make_async_copy(
                        obufs[b], out_hbm.at[pl.ds(0, no)], sos[b]
                    ).wait()

                _compute_chunk(xbufs[b], t_v, obufs[b], iota)
                pltpu.async_copy(obufs[b], out_hbm.at[pl.ds(base * _DOUT, no)], sos[b])

    pltpu.make_async_copy(ov1, out_hbm.at[pl.ds(0, no)], so1).wait()
    pltpu.make_async_copy(ov0, out_hbm.at[pl.ds(0, no)], so0).wait()


@jax.jit
def _run(xp_flat, t_flat):
    mesh = plsc.VectorSubcoreMesh(core_axis_name="c", subcore_axis_name="s")
    f = pl.kernel(
        _sc_body,
        mesh=mesh,
        compiler_params=pltpu.CompilerParams(needs_layout_passes=False),
        out_type=jax.ShapeDtypeStruct((_N * _DOUT,), jnp.float32),
        scratch_types=[
            pltpu.VMEM((_CHUNK * _XW,), jnp.float32),
            pltpu.VMEM((_CHUNK * _XW,), jnp.float32),
            pltpu.VMEM((_CHUNK * _DOUT,), jnp.float32),
            pltpu.VMEM((_CHUNK * _DOUT,), jnp.float32),
            pltpu.VMEM((_TROWS * _TS,), jnp.float32),
            pltpu.SemaphoreType.DMA,
            pltpu.SemaphoreType.DMA,
            pltpu.SemaphoreType.DMA,
            pltpu.SemaphoreType.DMA,
        ],
    )
    return f(xp_flat, t_flat)


def kernel(x, W0, W1, W2, W3, W4, W5, W6, W7, W8):
    Ws = [W0, W1, W2, W3, W4, W5, W6, W7, W8]
    secs = [W0]
    for a, b in _PAIRS:
        secs.append((Ws[a][:, None, :] + Ws[b][None, :, :]).reshape(-1, _D))
    table = jnp.concatenate(secs, axis=0)
    table = jnp.pad(table, ((0, 0), (0, _TS - _D)))
    xp = jnp.pad(x, ((0, 0), (0, _XW - x.shape[1])))
    out = _run(xp.reshape(-1), table.reshape(-1))
    return out.reshape(_N, _DOUT)


# speedup vs baseline: 3.2925x; 1.6105x over previous
import jax
import jax.numpy as jnp
from jax import lax
from jax.experimental import pallas as pl
from jax.experimental.pallas import tpu as pltpu
from jax.experimental.pallas import tpu_sc as plsc

_DIMS = [119, 5, 12, 12, 10, 6, 6, 2, 2]
_K = 8
_D = 248
_DOUT = 256
_N = 100000
_XW = 32
_XS = 33
_OS = 257
_TS = _D + 1

_NC = 2
_NS = 16
_NW = _NC * _NS
_RPW = _N // _NW
_CHUNK = 96
_NCHUNK = 33

_PAIRS = [(2, 8), (3, 7), (4, 1), (5, 6)]
_SECROWS = [_DIMS[0]] + [_DIMS[a] * _DIMS[b] for a, b in _PAIRS]
_SSTART = [0]
for _r in _SECROWS[:-1]:
    _SSTART.append(_SSTART[-1] + _r)
_TROWS = sum(_SECROWS)


def _compute_chunk(x_v, t_v, out_v, iota):
    zero = iota * 0
    for g in range(6):
        rb = g * 16
        rowv = iota + rb

        def gat(col):
            return plsc.load_gather(x_v, [rowv, zero + col])

        for c in range(_K):
            plsc.store_scatter(out_v, [rowv, zero + c], gat(c))

        ci = [gat(_K + i).astype(jnp.int32) for i in range(9)]
        addrs = [ci[0] * _TS]
        for p, (a, b) in enumerate(_PAIRS):
            addrs.append(
                ci[a] * (_DIMS[b] * _TS) + ci[b] * _TS + _SSTART[1 + p] * _TS
            )

        @plsc.parallel_loop(0, _D, unroll=8)
        def _(cc):
            v = None
            for k in range(5):
                gk = plsc.load_gather(t_v, [addrs[k] + cc])
                v = gk if v is None else v + gk
            plsc.store_scatter(out_v, [rowv, (zero + _K) + cc], v)


def _sc_body(x_hbm, t_hbm, out_hbm, xv0, xv1, ov0, ov1, t_v, sx0, sx1, so0, so1):
    wid = lax.axis_index("s") * _NC + lax.axis_index("c")
    pltpu.sync_copy(t_hbm, t_v)
    iota = lax.iota(jnp.int32, 16)
    w0 = (wid * _RPW) // 8 * 8
    xbufs = [xv0, xv1]
    obufs = [ov0, ov1]
    sxs = [sx0, sx1]
    sos = [so0, so1]

    def base_of(t):
        return jnp.minimum(w0 + t * _CHUNK, _N - _CHUNK)

    def x_src(t):
        return x_hbm.at[pl.ds(base_of(t), _CHUNK), pl.ds(0, _XW)]

    def x_dst(b):
        return xbufs[b].at[pl.ds(0, _CHUNK), pl.ds(0, _XW)]

    def o_src(b):
        return obufs[b].at[pl.ds(0, _CHUNK), pl.ds(0, _DOUT)]

    def o_dst(t):
        return out_hbm.at[pl.ds(base_of(t), _CHUNK), pl.ds(0, _DOUT)]

    pltpu.async_copy(x_src(0), x_dst(0), sx0)

    @pl.loop(0, _NCHUNK, step=2)
    def _(tt):
        for b in range(2):
            t = tt + b

            @pl.when(t < _NCHUNK)
            def _():
                pltpu.make_async_copy(x_src(t), x_dst(b), sxs[b]).wait()

                @pl.when(t + 1 < _NCHUNK)
                def _():
                    pltpu.async_copy(x_src(t + 1), x_dst(1 - b), sxs[1 - b])

                @pl.when(t >= 2)
                def _():
                    pltpu.make_async_copy(o_src(b), o_dst(0), sos[b]).wait()

                _compute_chunk(xbufs[b], t_v, obufs[b], iota)
                pltpu.async_copy(o_src(b), o_dst(t), sos[b])

    pltpu.make_async_copy(o_src(1), o_dst(0), so1).wait()
    pltpu.make_async_copy(o_src(0), o_dst(0), so0).wait()


@jax.jit
def _run(xp, t_flat):
    mesh = plsc.VectorSubcoreMesh(core_axis_name="c", subcore_axis_name="s")
    f = pl.kernel(
        _sc_body,
        mesh=mesh,
        compiler_params=pltpu.CompilerParams(
            needs_layout_passes=False, use_tc_tiling_on_sc=False
        ),
        out_type=jax.ShapeDtypeStruct((_N, _DOUT), jnp.float32),
        scratch_types=[
            pltpu.VMEM((_CHUNK, _XS), jnp.float32),
            pltpu.VMEM((_CHUNK, _XS), jnp.float32),
            pltpu.VMEM((_CHUNK, _OS), jnp.float32),
            pltpu.VMEM((_CHUNK, _OS), jnp.float32),
            pltpu.VMEM((_TROWS * _TS,), jnp.float32),
            pltpu.SemaphoreType.DMA,
            pltpu.SemaphoreType.DMA,
            pltpu.SemaphoreType.DMA,
            pltpu.SemaphoreType.DMA,
        ],
    )
    return f(xp, t_flat)


def kernel(x, W0, W1, W2, W3, W4, W5, W6, W7, W8):
    Ws = [W0, W1, W2, W3, W4, W5, W6, W7, W8]
    secs = [W0]
    for a, b in _PAIRS:
        secs.append((Ws[a][:, None, :] + Ws[b][None, :, :]).reshape(-1, _D))
    table = jnp.concatenate(secs, axis=0)
    table = jnp.pad(table, ((0, 0), (0, _TS - _D)))
    xp = jnp.pad(x, ((0, 0), (0, _XW - x.shape[1])))
    return _run(xp, table.reshape(-1))
